# Initial kernel scaffold; baseline (speedup 1.0000x reference)
import jax
import jax.numpy as jnp
from jax.experimental import pallas as pl


def kernel(x, edge_index, edge_weight, subG_node, W, b, Wp, bp):
    raise NotImplementedError("write your pallas kernel here")


# speedup vs baseline: 41.9978x; 41.9978x over previous
import functools

import jax
import jax.numpy as jnp
from jax import lax
from jax.experimental import pallas as pl
from jax.experimental.pallas import tpu as pltpu
from jax.experimental.pallas import tpu_sc as plsc

NC = 2
NS = 16
NW = NC * NS
L = 16


def _zmat_body(C, D, xr_ref, w_ref, wp_ref, b_ref, bp_ref, z_ref, c_ref):
    wv = jnp.dot(w_ref[...], wp_ref[...], preferred_element_type=jnp.float32)
    xm = xr_ref[:, 0:D]
    for c in range(1, C):
        xm = xm + xr_ref[:, c * D:(c + 1) * D]
    xm = xm * (1.0 / C)
    z_ref[...] = jnp.dot(xm, wv, preferred_element_type=jnp.float32)
    c_ref[...] = jnp.dot(b_ref[...], wv, preferred_element_type=jnp.float32) + bp_ref[...]


def _scatter_body(src_hbm, ew_hbm, dst_hbm, z_hbm, apart_hbm,
                  src_f, ew_f, msg_f, dst_v, z_v, buf_v, shared_a):
    cid = lax.axis_index("c")
    sid = lax.axis_index("s")
    wid = cid * NS + sid
    ept = src_f.shape[0]
    nch = dst_v.shape[0]
    slc = buf_v.shape[0]

    pltpu.sync_copy(src_hbm.at[wid], src_f)
    pltpu.sync_copy(ew_hbm.at[wid], ew_f)
    pltpu.sync_copy(dst_hbm.at[wid], dst_v)
    pltpu.sync_copy(z_hbm, z_v)

    def zero_loop(i, carry):
        buf_v[pl.ds(i * L, L)] = jnp.zeros((L,), jnp.float32)
        return carry

    lax.fori_loop(0, slc // L, zero_loop, 0)
    pltpu.sync_copy(buf_v, shared_a.at[pl.ds(sid * slc, slc)])
    plsc.subcore_barrier()

    def msg_loop(i, carry):
        s16 = src_f[pl.ds(i * L, L)]
        w16 = ew_f[pl.ds(i * L, L)]
        zg = plsc.load_gather(z_v, [s16])
        msg_f[pl.ds(i * L, L)] = zg * w16
        return carry

    lax.fori_loop(0, ept // L, msg_loop, 0)

    def scat_loop(j, carry):
        pltpu.sync_copy(msg_f.at[pl.ds(j * 128, 128)],
                        shared_a.at[dst_v.at[j]], add=True)
        return carry

    lax.fori_loop(0, nch, scat_loop, 0)
    plsc.subcore_barrier()

    pltpu.sync_copy(shared_a.at[pl.ds(sid * slc, slc)], buf_v)
    pltpu.sync_copy(buf_v, apart_hbm.at[wid])


def _gather_body(apart_hbm, i0_hbm, i1_hbm, c16_hbm, out_hbm,
                 a0_v, a1_v, i0_v, i1_v, o_v, c_v):
    cid = lax.axis_index("c")
    sid = lax.axis_index("s")
    wid = cid * NS + sid
    spt = i0_v.shape[0]

    pltpu.sync_copy(apart_hbm.at[0], a0_v)
    pltpu.sync_copy(apart_hbm.at[1], a1_v)
    pltpu.sync_copy(i0_hbm.at[wid], i0_v)
    pltpu.sync_copy(i1_hbm.at[wid], i1_v)
    pltpu.sync_copy(c16_hbm, c_v)
    cv = c_v[...]

    def g_loop(k, carry):
        x0 = i0_v[pl.ds(k * L, L)]
        x1 = i1_v[pl.ds(k * L, L)]
        g = (plsc.load_gather(a0_v, [x0]) + plsc.load_gather(a1_v, [x0])
             + plsc.load_gather(a0_v, [x1]) + plsc.load_gather(a1_v, [x1]))
        o_v[pl.ds(k * L, L)] = g * 0.5 + cv
        return carry

    lax.fori_loop(0, spt // L, g_loop, 0)
    pltpu.sync_copy(o_v, out_hbm.at[wid])


def kernel(x, edge_index, edge_weight, subG_node, W, b, Wp, bp):
    N, C, D = x.shape
    E = edge_index.shape[1]
    S, K = subG_node.shape
    assert K == 2 and S % (NW * L) == 0

    xr = x.reshape(N, C * D)
    z2, c2 = pl.pallas_call(
        functools.partial(_zmat_body, C, D),
        out_shape=(jax.ShapeDtypeStruct((N, 1), jnp.float32),
                   jax.ShapeDtypeStruct((1, 1), jnp.float32)),
    )(xr, W, Wp, b.reshape(1, D), bp.reshape(1, 1))
    c16 = jnp.broadcast_to(c2.reshape(1), (L,))

    ept = -(-E // (NW * 128)) * 128
    epad = ept * NW
    npad = -(-N // (NS * L)) * (NS * L)
    slc = npad // NS
    spt = S // NW

    z = jnp.concatenate([z2.reshape(N), jnp.zeros((npad - N,), jnp.float32)])

    pad = epad - E
    srcr = jnp.concatenate(
        [edge_index[0], jnp.zeros((pad,), jnp.int32)]).reshape(NW, ept)
    ewr = jnp.concatenate(
        [edge_weight, jnp.zeros((pad,), jnp.float32)]).reshape(NW, ept)
    dstr = jnp.concatenate(
        [edge_index[1], jnp.zeros((pad,), jnp.int32)]).reshape(NW, ept // 128, 128)

    mesh = plsc.VectorSubcoreMesh(core_axis_name="c", subcore_axis_name="s",
                                  num_cores=NC, num_subcores=NS)
    sc_params = pltpu.CompilerParams(needs_layout_passes=False)

    scatter = pl.kernel(
        _scatter_body,
        out_type=jax.ShapeDtypeStruct((NW, slc), jnp.float32),
        mesh=mesh,
        compiler_params=sc_params,
        scratch_types=[
            pltpu.VMEM((ept,), jnp.int32),
            pltpu.VMEM((ept,), jnp.float32),
            pltpu.VMEM((ept,), jnp.float32),
            pltpu.VMEM((ept // 128, 128), jnp.int32),
            pltpu.VMEM((npad,), jnp.float32),
            pltpu.VMEM((slc,), jnp.float32),
            pltpu.VMEM_SHARED((npad,), jnp.float32),
        ],
    )
    apart = scatter(srcr, ewr, dstr, z).reshape(NC, npad)

    i0 = subG_node[:, 0].reshape(NW, spt)
    i1 = subG_node[:, 1].reshape(NW, spt)

    gather = pl.kernel(
        _gather_body,
        out_type=jax.ShapeDtypeStruct((NW, spt), jnp.float32),
        mesh=mesh,
        compiler_params=sc_params,
        scratch_types=[
            pltpu.VMEM((npad,), jnp.float32),
            pltpu.VMEM((npad,), jnp.float32),
            pltpu.VMEM((spt,), jnp.int32),
            pltpu.VMEM((spt,), jnp.int32),
            pltpu.VMEM((spt,), jnp.float32),
            pltpu.VMEM((L,), jnp.float32),
        ],
    )
    out = gather(apart, i0, i1, c16)
    return out.reshape(S, 1)


# speedup vs baseline: 48.2007x; 1.1477x over previous
import functools

import jax
import jax.numpy as jnp
from jax import lax
from jax.experimental import pallas as pl
from jax.experimental.pallas import tpu as pltpu
from jax.experimental.pallas import tpu_sc as plsc

NC = 2
NS = 16
NW = NC * NS
L = 16
CW = 80


def _zmat_body(C, D, xr_ref, w_ref, wp_ref, b_ref, bp_ref, z_ref, c_ref):
    wv = jnp.dot(w_ref[...], wp_ref[...], preferred_element_type=jnp.float32)
    xm = xr_ref[:, 0:D]
    for c in range(1, C):
        xm = xm + xr_ref[:, c * D:(c + 1) * D]
    xm = xm * (1.0 / C)
    z_ref[...] = jnp.dot(xm, wv, preferred_element_type=jnp.float32)
    c_ref[...] = jnp.dot(b_ref[...], wv, preferred_element_type=jnp.float32) + bp_ref[...]


def _scatter_body(src_hbm, ew_hbm, dst_hbm, z_hbm, apart_hbm,
                  src_f, ew_f, msg_f, dst_v, z_v, buf_v, shared_a, sem, ssem):
    cid = lax.axis_index("c")
    sid = lax.axis_index("s")
    wid = cid * NS + sid
    nch = dst_v.shape[0]
    slc = buf_v.shape[0]

    cp1 = pltpu.async_copy(src_hbm.at[wid], src_f, sem)
    cp2 = pltpu.async_copy(ew_hbm.at[wid], ew_f, sem)
    cp3 = pltpu.async_copy(dst_hbm.at[wid], dst_v, sem)
    cp4 = pltpu.async_copy(z_hbm, z_v, sem)

    def zero_loop(i, carry):
        buf_v[pl.ds(i * L, L)] = jnp.zeros((L,), jnp.float32)
        return carry

    lax.fori_loop(0, slc // L, zero_loop, 0)
    cp1.wait()
    cp2.wait()
    cp3.wait()
    cp4.wait()
    pltpu.sync_copy(buf_v, shared_a.at[pl.ds(sid * slc, slc)])
    plsc.subcore_barrier()

    def chunk_loop(j, carry):
        for t in range(CW // L):
            off = j * CW + t * L
            s16 = src_f[pl.ds(off, L)]
            w16 = ew_f[pl.ds(off, L)]
            msg_f[pl.ds(off, L)] = plsc.load_gather(z_v, [s16]) * w16
        pltpu.async_copy(msg_f.at[pl.ds(j * CW, CW)],
                         shared_a.at[dst_v.at[j]], ssem, add=True)
        return carry

    lax.fori_loop(0, nch, chunk_loop, 0)

    def drain_loop(j, carry):
        pltpu.make_async_copy(msg_f.at[pl.ds(j * CW, CW)],
                              shared_a.at[dst_v.at[j]], ssem).wait()
        return carry

    lax.fori_loop(0, nch, drain_loop, 0)
    plsc.subcore_barrier()

    pltpu.sync_copy(shared_a.at[pl.ds(sid * slc, slc)], buf_v)
    pltpu.sync_copy(buf_v, apart_hbm.at[wid])


def _gather_body(apart_hbm, i0_hbm, i1_hbm, c16_hbm, out_hbm,
                 a0_v, a1_v, i0_v, i1_v, o_v, c_v, sem):
    cid = lax.axis_index("c")
    sid = lax.axis_index("s")
    wid = cid * NS + sid
    spt = i0_v.shape[0]

    cps = [pltpu.async_copy(apart_hbm.at[0], a0_v, sem),
           pltpu.async_copy(apart_hbm.at[1], a1_v, sem),
           pltpu.async_copy(i0_hbm.at[wid], i0_v, sem),
           pltpu.async_copy(i1_hbm.at[wid], i1_v, sem),
           pltpu.async_copy(c16_hbm, c_v, sem)]
    for cp in cps:
        cp.wait()
    cv = c_v[...]

    def g_loop(k, carry):
        x0 = i0_v[pl.ds(k * L, L)]
        x1 = i1_v[pl.ds(k * L, L)]
        g = (plsc.load_gather(a0_v, [x0]) + plsc.load_gather(a1_v, [x0])
             + plsc.load_gather(a0_v, [x1]) + plsc.load_gather(a1_v, [x1]))
        o_v[pl.ds(k * L, L)] = g * 0.5 + cv
        return carry

    lax.fori_loop(0, spt // L, g_loop, 0)
    pltpu.sync_copy(o_v, out_hbm.at[wid])


def kernel(x, edge_index, edge_weight, subG_node, W, b, Wp, bp):
    N, C, D = x.shape
    E = edge_index.shape[1]
    S, K = subG_node.shape
    ept = E // NW
    assert K == 2 and S % (NW * L) == 0 and E == ept * NW and ept % CW == 0

    xr = x.reshape(N, C * D)
    z2, c2 = pl.pallas_call(
        functools.partial(_zmat_body, C, D),
        out_shape=(jax.ShapeDtypeStruct((N, 1), jnp.float32),
                   jax.ShapeDtypeStruct((1, 1), jnp.float32)),
    )(xr, W, Wp, b.reshape(1, D), bp.reshape(1, 1))
    c16 = jnp.broadcast_to(c2.reshape(1), (L,))

    npad = -(-N // (NS * L)) * (NS * L)
    slc = npad // NS
    spt = S // NW
    z = jnp.concatenate([z2.reshape(N), jnp.zeros((npad - N,), jnp.float32)])

    srcr = edge_index[0].reshape(NW, ept)
    ewr = edge_weight.reshape(NW, ept)
    dstr = edge_index[1].reshape(NW, ept // CW, CW)

    mesh = plsc.VectorSubcoreMesh(core_axis_name="c", subcore_axis_name="s",
                                  num_cores=NC, num_subcores=NS)
    sc_params = pltpu.CompilerParams(needs_layout_passes=False)

    scatter = pl.kernel(
        _scatter_body,
        out_type=jax.ShapeDtypeStruct((NW, slc), jnp.float32),
        mesh=mesh,
        compiler_params=sc_params,
        scratch_types=[
            pltpu.VMEM((ept,), jnp.int32),
            pltpu.VMEM((ept,), jnp.float32),
            pltpu.VMEM((ept,), jnp.float32),
            pltpu.VMEM((ept // CW, CW), jnp.int32),
            pltpu.VMEM((npad,), jnp.float32),
            pltpu.VMEM((slc,), jnp.float32),
            pltpu.VMEM_SHARED((npad,), jnp.float32),
            pltpu.SemaphoreType.DMA,
            pltpu.SemaphoreType.DMA,
        ],
    )
    apart = scatter(srcr, ewr, dstr, z).reshape(NC, npad)

    i0 = subG_node[:, 0].reshape(NW, spt)
    i1 = subG_node[:, 1].reshape(NW, spt)

    gather = pl.kernel(
        _gather_body,
        out_type=jax.ShapeDtypeStruct((NW, spt), jnp.float32),
        mesh=mesh,
        compiler_params=sc_params,
        scratch_types=[
            pltpu.VMEM((npad,), jnp.float32),
            pltpu.VMEM((npad,), jnp.float32),
            pltpu.VMEM((spt,), jnp.int32),
            pltpu.VMEM((spt,), jnp.int32),
            pltpu.VMEM((spt,), jnp.float32),
            pltpu.VMEM((L,), jnp.float32),
            pltpu.SemaphoreType.DMA,
        ],
    )
    out = gather(apart, i0, i1, c16)
    return out.reshape(S, 1)


# speedup vs baseline: 52.3808x; 1.0867x over previous
import functools

import jax
import jax.numpy as jnp
from jax import lax
from jax.experimental import pallas as pl
from jax.experimental.pallas import tpu as pltpu
from jax.experimental.pallas import tpu_sc as plsc

NS = 16
L = 16


def _zmat_body(C, D, xr_ref, w_ref, wp_ref, b_ref, bp_ref, z_ref, c_ref):
    wv = jnp.dot(w_ref[...], wp_ref[...], preferred_element_type=jnp.float32)
    xm = xr_ref[:, 0:D]
    for c in range(1, C):
        xm = xm + xr_ref[:, c * D:(c + 1) * D]
    xm = xm * (1.0 / C)
    z_ref[...] = jnp.dot(xm, wv, preferred_element_type=jnp.float32)
    c_ref[...] = jnp.dot(b_ref[...], wv, preferred_element_type=jnp.float32) + bp_ref[...]


def _sc_body(src_hbm, ew_hbm, dst_hbm, z_hbm, i0_hbm, i1_hbm, c16_hbm, out_hbm,
             src_f, ew_f, msg_f, dst_f, z_v, buf_v, i0_v, i1_v, o_v, c_v,
             shared_a, sem):
    sid = lax.axis_index("s")
    ept = src_f.shape[0]
    slc = buf_v.shape[0]
    spt = i0_v.shape[0]
    ebase = sid * ept

    cps = [pltpu.async_copy(src_hbm.at[pl.ds(ebase, ept)], src_f, sem),
           pltpu.async_copy(ew_hbm.at[pl.ds(ebase, ept)], ew_f, sem),
           pltpu.async_copy(dst_hbm.at[pl.ds(ebase, ept)], dst_f, sem),
           pltpu.async_copy(z_hbm, z_v, sem)]

    def zero_loop(i, carry):
        buf_v[pl.ds(i * L, L)] = jnp.zeros((L,), jnp.float32)
        return carry

    lax.fori_loop(0, slc // L, zero_loop, 0)
    for cp in cps:
        cp.wait()
    pltpu.sync_copy(buf_v, shared_a.at[pl.ds(sid * slc, slc)])
    plsc.subcore_barrier()

    def msg_loop(i, carry):
        s16 = src_f[pl.ds(i * L, L)]
        w16 = ew_f[pl.ds(i * L, L)]
        msg_f[pl.ds(i * L, L)] = plsc.load_gather(z_v, [s16]) * w16
        return carry

    lax.fori_loop(0, ept // L, msg_loop, 0)

    pltpu.sync_copy(msg_f, shared_a.at[dst_f], add=True)
    plsc.subcore_barrier()

    cps = [pltpu.async_copy(i0_hbm.at[pl.ds(sid * spt, spt)], i0_v, sem),
           pltpu.async_copy(i1_hbm.at[pl.ds(sid * spt, spt)], i1_v, sem),
           pltpu.async_copy(c16_hbm, c_v, sem)]
    pltpu.sync_copy(shared_a, z_v)
    for cp in cps:
        cp.wait()
    cv = c_v[...]

    def g_loop(k, carry):
        x0 = i0_v[pl.ds(k * L, L)]
        x1 = i1_v[pl.ds(k * L, L)]
        g = plsc.load_gather(z_v, [x0]) + plsc.load_gather(z_v, [x1])
        o_v[pl.ds(k * L, L)] = g * 0.5 + cv
        return carry

    lax.fori_loop(0, spt // L, g_loop, 0)
    pltpu.sync_copy(o_v, out_hbm.at[pl.ds(sid * spt, spt)])


def kernel(x, edge_index, edge_weight, subG_node, W, b, Wp, bp):
    N, C, D = x.shape
    E = edge_index.shape[1]
    S, K = subG_node.shape
    ept = E // NS
    spt = S // NS
    assert K == 2 and S % (NS * L) == 0 and E == ept * NS and ept % L == 0

    xr = x.reshape(N, C * D)
    z2, c2 = pl.pallas_call(
        functools.partial(_zmat_body, C, D),
        out_shape=(jax.ShapeDtypeStruct((N, 1), jnp.float32),
                   jax.ShapeDtypeStruct((1, 1), jnp.float32)),
    )(xr, W, Wp, b.reshape(1, D), bp.reshape(1, 1))
    c16 = jnp.broadcast_to(c2.reshape(1), (L,))

    npad = -(-N // (NS * L)) * (NS * L)
    slc = npad // NS
    z = jnp.concatenate([z2.reshape(N), jnp.zeros((npad - N,), jnp.float32)])

    src = edge_index[0]
    dst = edge_index[1]
    i0 = subG_node[:, 0]
    i1 = subG_node[:, 1]

    mesh = plsc.VectorSubcoreMesh(core_axis_name="c", subcore_axis_name="s",
                                  num_cores=1, num_subcores=NS)
    sc_params = pltpu.CompilerParams(needs_layout_passes=False)

    sc = pl.kernel(
        _sc_body,
        out_type=jax.ShapeDtypeStruct((S,), jnp.float32),
        mesh=mesh,
        compiler_params=sc_params,
        scratch_types=[
            pltpu.VMEM((ept,), jnp.int32),
            pltpu.VMEM((ept,), jnp.float32),
            pltpu.VMEM((ept,), jnp.float32),
            pltpu.VMEM((ept,), jnp.int32),
            pltpu.VMEM((npad,), jnp.float32),
            pltpu.VMEM((slc,), jnp.float32),
            pltpu.VMEM((spt,), jnp.int32),
            pltpu.VMEM((spt,), jnp.int32),
            pltpu.VMEM((spt,), jnp.float32),
            pltpu.VMEM((L,), jnp.float32),
            pltpu.VMEM_SHARED((npad,), jnp.float32),
            pltpu.SemaphoreType.DMA,
        ],
    )
    out = sc(src, edge_weight, dst, z, i0, i1, c16)
    return out.reshape(S, 1)
